# baseline (speedup 1.0000x reference)
import functools

import jax
import jax.numpy as jnp
from jax import lax
from jax.experimental import pallas as pl
from jax.experimental.pallas import tpu as pltpu
from jax.experimental.pallas import tpu_sc as plsc

_B, _C, _H, _W = 16, 64, 32, 32
_HW = _H * _W
_K = 1024
_N = _B * _HW
_COMMIT = 0.25

_NC, _NS, _L = 2, 16, 16
_NWORK = _NC * _NS
_TOK = _N // _NWORK
_TV = _TOK // _L


def _argmin_body(x_ref, w_ref, b_ref, emb_ref, emb2_ref,
                 idx_ref, loss_ref, dsum_acc):
    b = pl.program_id(0)
    x = x_ref[0]
    w = w_ref[...]
    emb = emb_ref[...]

    z = jax.lax.dot_general(w, x, (((1,), (0,)), ((), ()))) + b_ref[...]

    zsq = jnp.sum(z * z, axis=0, keepdims=True)
    esq = jnp.sum(emb * emb, axis=1, keepdims=True)
    s2 = jax.lax.dot_general(emb2_ref[...], z, (((1,), (0,)), ((), ())))
    dist = (zsq + esq) - s2

    m = jnp.min(dist, axis=0, keepdims=True)
    idx_ref[0] = jnp.argmin(dist, axis=0)[None, :]

    dsum_b = jnp.sum(m)

    @pl.when(b == 0)
    def _init():
        dsum_acc[0] = dsum_b

    @pl.when(b > 0)
    def _acc():
        dsum_acc[0] = dsum_acc[0] + dsum_b

    @pl.when(b == _B - 1)
    def _fin():
        loss = (1.0 + _COMMIT) * dsum_acc[0] / float(_N * _C)
        loss_ref[...] = jnp.broadcast_to(loss, (1, 1))


def _tc_argmin(xr, W_lin, br, emb):
    return pl.pallas_call(
        _argmin_body,
        grid=(_B,),
        in_specs=[
            pl.BlockSpec((1, _C, _HW), lambda b: (b, 0, 0)),
            pl.BlockSpec((_C, _C), lambda b: (0, 0)),
            pl.BlockSpec((_C, 1), lambda b: (0, 0)),
            pl.BlockSpec((_K, _C), lambda b: (0, 0)),
            pl.BlockSpec((_K, _C), lambda b: (0, 0)),
        ],
        out_specs=[
            pl.BlockSpec((1, 1, _HW), lambda b: (b, 0, 0)),
            pl.BlockSpec((1, 1), lambda b: (0, 0)),
        ],
        out_shape=[
            jax.ShapeDtypeStruct((_B, 1, _HW), jnp.int32),
            jax.ShapeDtypeStruct((1, 1), jnp.float32),
        ],
        scratch_shapes=[pltpu.SMEM((1,), jnp.float32)],
    )(xr, W_lin, br, emb, emb * 2.0)


_SC_MESH = plsc.VectorSubcoreMesh(
    core_axis_name="c", subcore_axis_name="s",
    num_cores=_NC, num_subcores=_NS)


@functools.partial(
    pl.kernel,
    out_type=[
        jax.ShapeDtypeStruct((_B, _C, _HW), jnp.float32),
        jax.ShapeDtypeStruct((_NC * _K,), jnp.float32),
    ],
    mesh=_SC_MESH,
    compiler_params=pltpu.CompilerParams(needs_layout_passes=False),
    scratch_types=[
        pltpu.VMEM((_C * _K,), jnp.float32),
        pltpu.VMEM((_TOK,), jnp.int32),
        pltpu.VMEM((_C, _TOK), jnp.float32),
        pltpu.VMEM((_K,), jnp.float32),
        pltpu.VMEM((_NS, 128), jnp.float32),
        pltpu.VMEM((128,), jnp.float32),
        pltpu.VMEM_SHARED((_NS, _K), jnp.float32),
    ])
def _sc_gather_hist(embT_hbm, idx_hbm, q_hbm, counts_hbm,
                    embT_v, idx_v, out_v, bins_v, red_v, cnt_v, shared):
    cid = lax.axis_index("c")
    sid = lax.axis_index("s")
    wid = cid * _NS + sid
    b = wid // 2
    h = wid % 2
    base = b * _HW + h * _TOK

    pltpu.sync_copy(embT_hbm, embT_v)
    pltpu.sync_copy(idx_hbm.at[pl.ds(base, _TOK)], idx_v)

    zeros = jnp.zeros((_L,), jnp.float32)
    ones = jnp.ones((_L,), jnp.float32)

    def _zero_bins(i, carry):
        bins_v[pl.ds(i * _L, _L)] = zeros
        return carry
    lax.fori_loop(0, _K // _L, _zero_bins, 0)

    def _gather_c(c, carry):
        coff = c * _K
        for v in range(_TV):
            iv = idx_v[pl.ds(v * _L, _L)]
            g = plsc.load_gather(embT_v, [iv + coff])
            out_v[c, pl.ds(v * _L, _L)] = g
        return carry
    lax.fori_loop(0, _C, _gather_c, 0)

    pltpu.sync_copy(out_v, q_hbm.at[b, :, pl.ds(h * _TOK, _TOK)])

    def _hist(v, carry):
        iv = idx_v[pl.ds(v * _L, _L)]
        plsc.addupdate_scatter(bins_v, [iv], ones)
        return carry
    lax.fori_loop(0, _TV, _hist, 0)

    pltpu.sync_copy(bins_v, shared.at[sid])
    plsc.subcore_barrier()

    @pl.when(sid < 8)
    def _reduce_slice():
        pltpu.sync_copy(shared.at[:, pl.ds(sid * 128, 128)], red_v)

        for v in range(128 // _L):
            cnt_v[pl.ds(v * _L, _L)] = red_v[0, pl.ds(v * _L, _L)]

        def _red(r, carry):
            for v in range(128 // _L):
                sl = pl.ds(v * _L, _L)
                cnt_v[sl] = cnt_v[sl] + red_v[r, sl]
            return carry
        lax.fori_loop(1, _NS, _red, 0)

        pltpu.sync_copy(cnt_v, counts_hbm.at[pl.ds(cid * _K + sid * 128, 128)])


def _perp_body(c2_ref, perp_ref):
    c2 = c2_ref[...]
    p = (c2[:, :_K] + c2[:, _K:]) * (1.0 / float(_N))
    perp = jnp.exp(-jnp.sum(p * jnp.log(p + 1e-10)))
    perp_ref[...] = jnp.broadcast_to(perp, (1, 1))


def _tc_perplexity(counts):
    return pl.pallas_call(
        _perp_body,
        out_shape=jax.ShapeDtypeStruct((1, 1), jnp.float32),
    )(counts.reshape(1, _NC * _K))


@jax.jit
def kernel(x, W_lin, b_lin, emb):
    xr = x.reshape(_B, _C, _HW)
    br = b_lin.reshape(_C, 1)
    idx_o, loss = _tc_argmin(xr, W_lin, br, emb)
    idx_flat = idx_o.reshape(_N)
    embT_flat = emb.T.reshape(_C * _K)
    q, counts = _sc_gather_hist(embT_flat, idx_flat)
    perp = _tc_perplexity(counts)
    return (loss[0, 0],
            q.reshape(_B, _C, _H, _W),
            perp[0, 0],
            idx_flat[:, None])


# speedup vs baseline: 1.2353x; 1.2353x over previous
import functools

import jax
import jax.numpy as jnp
from jax import lax
from jax.experimental import pallas as pl
from jax.experimental.pallas import tpu as pltpu
from jax.experimental.pallas import tpu_sc as plsc

_B, _C, _H, _W = 16, 64, 32, 32
_HW = _H * _W
_K = 1024
_N = _B * _HW
_COMMIT = 0.25

_NC, _NS, _L = 2, 16, 16
_NWORK = _NC * _NS
_TOK = _N // _NWORK
_TV = _TOK // _L


def _argmin_body(x_ref, w_ref, b_ref, emb_ref, emb2_ref,
                 idx_ref, loss_ref, dsum_acc):
    b = pl.program_id(0)
    x = x_ref[0]
    w = w_ref[...]
    emb = emb_ref[...]

    z = jax.lax.dot_general(w, x, (((1,), (0,)), ((), ()))) + b_ref[...]

    zsq = jnp.sum(z * z, axis=0, keepdims=True)
    esq = jnp.sum(emb * emb, axis=1, keepdims=True)
    s2 = jax.lax.dot_general(emb2_ref[...], z, (((1,), (0,)), ((), ())))
    dist = (zsq + esq) - s2

    m = jnp.min(dist, axis=0, keepdims=True)
    idx_ref[0] = jnp.argmin(dist, axis=0)[None, :]

    dsum_b = jnp.sum(m)

    @pl.when(b == 0)
    def _init():
        dsum_acc[0] = dsum_b

    @pl.when(b > 0)
    def _acc():
        dsum_acc[0] = dsum_acc[0] + dsum_b

    @pl.when(b == _B - 1)
    def _fin():
        loss = (1.0 + _COMMIT) * dsum_acc[0] / float(_N * _C)
        loss_ref[...] = jnp.broadcast_to(loss, (1, 1))


def _tc_argmin(xr, W_lin, br, emb):
    return pl.pallas_call(
        _argmin_body,
        grid=(_B,),
        in_specs=[
            pl.BlockSpec((1, _C, _HW), lambda b: (b, 0, 0)),
            pl.BlockSpec((_C, _C), lambda b: (0, 0)),
            pl.BlockSpec((_C, 1), lambda b: (0, 0)),
            pl.BlockSpec((_K, _C), lambda b: (0, 0)),
            pl.BlockSpec((_K, _C), lambda b: (0, 0)),
        ],
        out_specs=[
            pl.BlockSpec((1, 1, _HW), lambda b: (b, 0, 0)),
            pl.BlockSpec((1, 1), lambda b: (0, 0)),
        ],
        out_shape=[
            jax.ShapeDtypeStruct((_B, 1, _HW), jnp.int32),
            jax.ShapeDtypeStruct((1, 1), jnp.float32),
        ],
        scratch_shapes=[pltpu.SMEM((1,), jnp.float32)],
    )(xr, W_lin, br, emb, emb * 2.0)


_SC_MESH = plsc.VectorSubcoreMesh(
    core_axis_name="c", subcore_axis_name="s",
    num_cores=_NC, num_subcores=_NS)


@functools.partial(
    pl.kernel,
    out_type=[
        jax.ShapeDtypeStruct((_B, _C, _HW), jnp.float32),
        jax.ShapeDtypeStruct((_NC * _K,), jnp.float32),
    ],
    mesh=_SC_MESH,
    compiler_params=pltpu.CompilerParams(needs_layout_passes=False),
    scratch_types=[
        pltpu.VMEM((_C * _K,), jnp.float32),
        pltpu.VMEM((_TOK,), jnp.int32),
        pltpu.VMEM((_C, _TOK), jnp.float32),
        pltpu.VMEM((_K,), jnp.float32),
        pltpu.VMEM((_NS, 128), jnp.float32),
        pltpu.VMEM((128,), jnp.float32),
        pltpu.VMEM_SHARED((_NS, _K), jnp.float32),
    ])
def _sc_gather_hist(embT_hbm, idx_hbm, q_hbm, counts_hbm,
                    embT_v, idx_v, out_v, bins_v, red_v, cnt_v, shared):
    cid = lax.axis_index("c")
    sid = lax.axis_index("s")
    wid = cid * _NS + sid
    b = wid // 2
    h = wid % 2
    base = b * _HW + h * _TOK

    pltpu.sync_copy(embT_hbm, embT_v)
    pltpu.sync_copy(idx_hbm.at[pl.ds(base, _TOK)], idx_v)

    zeros = jnp.zeros((_L,), jnp.float32)
    ones = jnp.ones((_L,), jnp.float32)

    def _zero_bins(i, carry):
        bins_v[pl.ds(i * _L, _L)] = zeros
        return carry
    lax.fori_loop(0, _K // _L, _zero_bins, 0)

    @plsc.parallel_loop(0, _TV, carry=jnp.int32(0))
    def _gather_v(v, carry):
        off = v * _L
        iv = idx_v[pl.ds(off, _L)]
        for c in range(_C):
            g = plsc.load_gather(embT_v, [iv + c * _K])
            out_v[c, pl.ds(off, _L)] = g
        return carry

    pltpu.sync_copy(out_v, q_hbm.at[b, :, pl.ds(h * _TOK, _TOK)])

    def _hist(v, carry):
        iv = idx_v[pl.ds(v * _L, _L)]
        plsc.addupdate_scatter(bins_v, [iv], ones)
        return carry
    lax.fori_loop(0, _TV, _hist, 0)

    pltpu.sync_copy(bins_v, shared.at[sid])
    plsc.subcore_barrier()

    @pl.when(sid < 8)
    def _reduce_slice():
        pltpu.sync_copy(shared.at[:, pl.ds(sid * 128, 128)], red_v)

        for v in range(128 // _L):
            cnt_v[pl.ds(v * _L, _L)] = red_v[0, pl.ds(v * _L, _L)]

        def _red(r, carry):
            for v in range(128 // _L):
                sl = pl.ds(v * _L, _L)
                cnt_v[sl] = cnt_v[sl] + red_v[r, sl]
            return carry
        lax.fori_loop(1, _NS, _red, 0)

        pltpu.sync_copy(cnt_v, counts_hbm.at[pl.ds(cid * _K + sid * 128, 128)])


def _perp_body(c2_ref, perp_ref):
    c2 = c2_ref[...]
    p = (c2[:, :_K] + c2[:, _K:]) * (1.0 / float(_N))
    perp = jnp.exp(-jnp.sum(p * jnp.log(p + 1e-10)))
    perp_ref[...] = jnp.broadcast_to(perp, (1, 1))


def _tc_perplexity(counts):
    return pl.pallas_call(
        _perp_body,
        out_shape=jax.ShapeDtypeStruct((1, 1), jnp.float32),
    )(counts.reshape(1, _NC * _K))


@jax.jit
def kernel(x, W_lin, b_lin, emb):
    xr = x.reshape(_B, _C, _HW)
    br = b_lin.reshape(_C, 1)
    idx_o, loss = _tc_argmin(xr, W_lin, br, emb)
    idx_flat = idx_o.reshape(_N)
    embT_flat = emb.T.reshape(_C * _K)
    q, counts = _sc_gather_hist(embT_flat, idx_flat)
    perp = _tc_perplexity(counts)
    return (loss[0, 0],
            q.reshape(_B, _C, _H, _W),
            perp[0, 0],
            idx_flat[:, None])


# speedup vs baseline: 1.2859x; 1.0410x over previous
import functools

import jax
import jax.numpy as jnp
from jax import lax
from jax.experimental import pallas as pl
from jax.experimental.pallas import tpu as pltpu
from jax.experimental.pallas import tpu_sc as plsc

_B, _C, _H, _W = 16, 64, 32, 32
_HW = _H * _W
_K = 1024
_N = _B * _HW
_COMMIT = 0.25

_NC, _NS, _L = 2, 16, 16
_NWORK = _NC * _NS
_TOK = _N // _NWORK
_TV = _TOK // _L


_BPS = 4
_STEPS = _B // _BPS


def _argmin_body(x_ref, w_ref, b_ref, emb_ref, emb2_ref,
                 idx_ref, loss_ref, dsum_acc):
    g = pl.program_id(0)
    w = w_ref[...]
    emb = emb_ref[...]
    emb2 = emb2_ref[...]
    bias = b_ref[...]
    esq = jnp.sum(emb * emb, axis=1, keepdims=True)

    dsum_g = jnp.float32(0.0)
    for i in range(_BPS):
        x = x_ref[i]
        z = jax.lax.dot_general(w, x, (((1,), (0,)), ((), ()))) + bias
        zsq = jnp.sum(z * z, axis=0, keepdims=True)
        s2 = jax.lax.dot_general(emb2, z, (((1,), (0,)), ((), ())))
        dist = (zsq + esq) - s2
        m = jnp.min(dist, axis=0, keepdims=True)
        idx_ref[i] = jnp.argmin(dist, axis=0)[None, :]
        dsum_g = dsum_g + jnp.sum(m)

    @pl.when(g == 0)
    def _init():
        dsum_acc[0] = dsum_g

    @pl.when(g > 0)
    def _acc():
        dsum_acc[0] = dsum_acc[0] + dsum_g

    @pl.when(g == _STEPS - 1)
    def _fin():
        loss = (1.0 + _COMMIT) * dsum_acc[0] / float(_N * _C)
        loss_ref[...] = jnp.broadcast_to(loss, (1, 1))


def _tc_argmin(xr, W_lin, br, emb):
    return pl.pallas_call(
        _argmin_body,
        grid=(_STEPS,),
        in_specs=[
            pl.BlockSpec((_BPS, _C, _HW), lambda g: (g, 0, 0)),
            pl.BlockSpec((_C, _C), lambda g: (0, 0)),
            pl.BlockSpec((_C, 1), lambda g: (0, 0)),
            pl.BlockSpec((_K, _C), lambda g: (0, 0)),
            pl.BlockSpec((_K, _C), lambda g: (0, 0)),
        ],
        out_specs=[
            pl.BlockSpec((_BPS, 1, _HW), lambda g: (g, 0, 0)),
            pl.BlockSpec((1, 1), lambda g: (0, 0)),
        ],
        out_shape=[
            jax.ShapeDtypeStruct((_B, 1, _HW), jnp.int32),
            jax.ShapeDtypeStruct((1, 1), jnp.float32),
        ],
        scratch_shapes=[pltpu.SMEM((1,), jnp.float32)],
    )(xr, W_lin, br, emb, emb * 2.0)


_SC_MESH = plsc.VectorSubcoreMesh(
    core_axis_name="c", subcore_axis_name="s",
    num_cores=_NC, num_subcores=_NS)


@functools.partial(
    pl.kernel,
    out_type=[
        jax.ShapeDtypeStruct((_B, _C, _HW), jnp.float32),
        jax.ShapeDtypeStruct((_NC * _K,), jnp.float32),
    ],
    mesh=_SC_MESH,
    compiler_params=pltpu.CompilerParams(needs_layout_passes=False),
    scratch_types=[
        pltpu.VMEM((_C * _K,), jnp.float32),
        pltpu.VMEM((_TOK,), jnp.int32),
        pltpu.VMEM((_C, _TOK), jnp.float32),
        pltpu.VMEM((_K,), jnp.float32),
        pltpu.VMEM((_NS, 128), jnp.float32),
        pltpu.VMEM((128,), jnp.float32),
        pltpu.VMEM_SHARED((_NS, _K), jnp.float32),
    ])
def _sc_gather_hist(embT_hbm, idx_hbm, q_hbm, counts_hbm,
                    embT_v, idx_v, out_v, bins_v, red_v, cnt_v, shared):
    cid = lax.axis_index("c")
    sid = lax.axis_index("s")
    wid = cid * _NS + sid
    b = wid // 2
    h = wid % 2
    base = b * _HW + h * _TOK

    pltpu.sync_copy(embT_hbm, embT_v)
    pltpu.sync_copy(idx_hbm.at[pl.ds(base, _TOK)], idx_v)

    zeros = jnp.zeros((_L,), jnp.float32)
    ones = jnp.ones((_L,), jnp.float32)

    def _zero_bins(i, carry):
        bins_v[pl.ds(i * _L, _L)] = zeros
        return carry
    lax.fori_loop(0, _K // _L, _zero_bins, 0)

    @plsc.parallel_loop(0, _TV, carry=jnp.int32(0))
    def _gather_v(v, carry):
        off = v * _L
        iv = idx_v[pl.ds(off, _L)]
        for c in range(_C):
            g = plsc.load_gather(embT_v, [iv + c * _K])
            out_v[c, pl.ds(off, _L)] = g
        return carry

    pltpu.sync_copy(out_v, q_hbm.at[b, :, pl.ds(h * _TOK, _TOK)])

    def _hist(v, carry):
        iv = idx_v[pl.ds(v * _L, _L)]
        plsc.addupdate_scatter(bins_v, [iv], ones)
        return carry
    lax.fori_loop(0, _TV, _hist, 0)

    pltpu.sync_copy(bins_v, shared.at[sid])
    plsc.subcore_barrier()

    @pl.when(sid < 8)
    def _reduce_slice():
        pltpu.sync_copy(shared.at[:, pl.ds(sid * 128, 128)], red_v)

        for v in range(128 // _L):
            cnt_v[pl.ds(v * _L, _L)] = red_v[0, pl.ds(v * _L, _L)]

        def _red(r, carry):
            for v in range(128 // _L):
                sl = pl.ds(v * _L, _L)
                cnt_v[sl] = cnt_v[sl] + red_v[r, sl]
            return carry
        lax.fori_loop(1, _NS, _red, 0)

        pltpu.sync_copy(cnt_v, counts_hbm.at[pl.ds(cid * _K + sid * 128, 128)])


def _perp_body(c2_ref, perp_ref):
    c2 = c2_ref[...]
    p = (c2[:, :_K] + c2[:, _K:]) * (1.0 / float(_N))
    perp = jnp.exp(-jnp.sum(p * jnp.log(p + 1e-10)))
    perp_ref[...] = jnp.broadcast_to(perp, (1, 1))


def _tc_perplexity(counts):
    return pl.pallas_call(
        _perp_body,
        out_shape=jax.ShapeDtypeStruct((1, 1), jnp.float32),
    )(counts.reshape(1, _NC * _K))


@jax.jit
def kernel(x, W_lin, b_lin, emb):
    xr = x.reshape(_B, _C, _HW)
    br = b_lin.reshape(_C, 1)
    idx_o, loss = _tc_argmin(xr, W_lin, br, emb)
    idx_flat = idx_o.reshape(_N)
    embT_flat = emb.T.reshape(_C * _K)
    q, counts = _sc_gather_hist(embT_flat, idx_flat)
    perp = _tc_perplexity(counts)
    return (loss[0, 0],
            q.reshape(_B, _C, _H, _W),
            perp[0, 0],
            idx_flat[:, None])


# speedup vs baseline: 1.2971x; 1.0087x over previous
import functools

import jax
import jax.numpy as jnp
from jax import lax
from jax.experimental import pallas as pl
from jax.experimental.pallas import tpu as pltpu
from jax.experimental.pallas import tpu_sc as plsc

_B, _C, _H, _W = 16, 64, 32, 32
_HW = _H * _W
_K = 1024
_N = _B * _HW
_COMMIT = 0.25

_NC, _NS, _L = 2, 16, 16
_NWORK = _NC * _NS
_VPB = _HW // _L


_BPS = 4
_STEPS = _B // _BPS


def _argmin_body(x_ref, w_ref, b_ref, emb_ref, emb2_ref,
                 idx_ref, loss_ref, perp_ref, dsum_acc, counts_acc):
    g = pl.program_id(0)
    w = w_ref[...]
    emb = emb_ref[...]
    emb2 = emb2_ref[...]
    bias = b_ref[...]
    esq = jnp.sum(emb * emb, axis=1, keepdims=True)
    kiota = jax.lax.broadcasted_iota(jnp.int32, (_K, _HW), 0)

    dsum_g = jnp.float32(0.0)
    cnt_g = jnp.zeros((_K, 1), jnp.float32)
    for i in range(_BPS):
        x = x_ref[i]
        z = jax.lax.dot_general(w, x, (((1,), (0,)), ((), ()))) + bias
        zsq = jnp.sum(z * z, axis=0, keepdims=True)
        s2 = jax.lax.dot_general(emb2, z, (((1,), (0,)), ((), ())))
        dist = (zsq + esq) - s2
        m = jnp.min(dist, axis=0, keepdims=True)
        idxr = jnp.argmin(dist, axis=0)[None, :]
        idx_ref[i] = idxr
        dsum_g = dsum_g + jnp.sum(m)
        onehot = (kiota == idxr).astype(jnp.float32)
        cnt_g = cnt_g + jnp.sum(onehot, axis=1, keepdims=True)

    @pl.when(g == 0)
    def _init():
        dsum_acc[0] = dsum_g
        counts_acc[...] = cnt_g

    @pl.when(g > 0)
    def _acc():
        dsum_acc[0] = dsum_acc[0] + dsum_g
        counts_acc[...] = counts_acc[...] + cnt_g

    @pl.when(g == _STEPS - 1)
    def _fin():
        loss = (1.0 + _COMMIT) * dsum_acc[0] / float(_N * _C)
        loss_ref[...] = jnp.broadcast_to(loss, (1, 1))
        p = counts_acc[...] * (1.0 / float(_N))
        perp = jnp.exp(-jnp.sum(p * jnp.log(p + 1e-10)))
        perp_ref[...] = jnp.broadcast_to(perp, (1, 1))


def _tc_argmin(xr, W_lin, br, emb):
    return pl.pallas_call(
        _argmin_body,
        grid=(_STEPS,),
        in_specs=[
            pl.BlockSpec((_BPS, _C, _HW), lambda g: (g, 0, 0)),
            pl.BlockSpec((_C, _C), lambda g: (0, 0)),
            pl.BlockSpec((_C, 1), lambda g: (0, 0)),
            pl.BlockSpec((_K, _C), lambda g: (0, 0)),
            pl.BlockSpec((_K, _C), lambda g: (0, 0)),
        ],
        out_specs=[
            pl.BlockSpec((_BPS, 1, _HW), lambda g: (g, 0, 0)),
            pl.BlockSpec((1, 1), lambda g: (0, 0)),
            pl.BlockSpec((1, 1), lambda g: (0, 0)),
        ],
        out_shape=[
            jax.ShapeDtypeStruct((_B, 1, _HW), jnp.int32),
            jax.ShapeDtypeStruct((1, 1), jnp.float32),
            jax.ShapeDtypeStruct((1, 1), jnp.float32),
        ],
        scratch_shapes=[pltpu.SMEM((1,), jnp.float32),
                        pltpu.VMEM((_K, 1), jnp.float32)],
    )(xr, W_lin, br, emb, emb * 2.0)


_SC_MESH = plsc.VectorSubcoreMesh(
    core_axis_name="c", subcore_axis_name="s",
    num_cores=_NC, num_subcores=_NS)


@functools.partial(
    pl.kernel,
    out_type=jax.ShapeDtypeStruct((_B, _C, _HW), jnp.float32),
    mesh=_SC_MESH,
    compiler_params=pltpu.CompilerParams(needs_layout_passes=False),
    scratch_types=[
        pltpu.VMEM((2 * _K,), jnp.float32),
        pltpu.VMEM((_N,), jnp.int32),
        pltpu.VMEM((_B, 2, _HW), jnp.float32),
    ])
def _sc_gather(embT_hbm, idx_hbm, q_hbm, tab_v, idx_v, out_v):
    cid = lax.axis_index("c")
    sid = lax.axis_index("s")
    wid = cid * _NS + sid
    c0 = wid * 2

    pltpu.sync_copy(embT_hbm.at[pl.ds(c0 * _K, 2 * _K)], tab_v)
    pltpu.sync_copy(idx_hbm, idx_v)

    @plsc.parallel_loop(0, _N // _L, carry=jnp.int32(0))
    def _g(v, carry):
        iv = idx_v[pl.ds(v * _L, _L)]
        b = v // _VPB
        off = (v % _VPB) * _L
        out_v[b, 0, pl.ds(off, _L)] = plsc.load_gather(tab_v, [iv])
        out_v[b, 1, pl.ds(off, _L)] = plsc.load_gather(tab_v, [iv + _K])
        return carry

    pltpu.sync_copy(out_v, q_hbm.at[:, pl.ds(c0, 2), :])


@jax.jit
def kernel(x, W_lin, b_lin, emb):
    xr = x.reshape(_B, _C, _HW)
    br = b_lin.reshape(_C, 1)
    idx_o, loss, perp = _tc_argmin(xr, W_lin, br, emb)
    idx_flat = idx_o.reshape(_N)
    embT_flat = emb.T.reshape(_C * _K)
    q = _sc_gather(embT_flat, idx_flat)
    return (loss[0, 0],
            q.reshape(_B, _C, _H, _W),
            perp[0, 0],
            idx_flat[:, None])
